# baseline (speedup 1.0000x reference)
import functools

import jax
import jax.numpy as jnp
from jax import lax
from jax.experimental import pallas as pl
from jax.experimental.pallas import tpu as pltpu
from jax.experimental.pallas import tpu_sc as plsc

N = 10000
D = 128
E = 320000

NC = 2
NS = 16
NW = NC * NS
EPW = E // NW
K = 40
NCHUNK = EPW // K
NPAD = 10240
RPT = NPAD // NS
DRPT = NPAD // NS
DW = 16
KD = 80
NCHUNKD = EPW // KD

_mesh = plsc.VectorSubcoreMesh(core_axis_name="c", subcore_axis_name="s")


@functools.partial(
    pl.kernel,
    mesh=_mesh,
    out_type=jax.ShapeDtypeStruct((NC, NPAD, DW), jnp.float32),
    scratch_types=[
        pltpu.VMEM_SHARED((NPAD, DW), jnp.float32),
        pltpu.VMEM((NCHUNKD, KD), jnp.int32),
        pltpu.VMEM((KD, DW), jnp.float32),
        pltpu.SemaphoreType.DMA((4,)),
    ],
    compiler_params=pltpu.CompilerParams(use_tc_tiling_on_sc=False,
                                         skip_device_barrier=True),
)
def _sc_degree(dst_hbm, ones_hbm, zeros_hbm, out_hbm, dacc, dst_v, ones_v,
               ssem):
    c = lax.axis_index("c")
    s = lax.axis_index("s")
    wid = s * NC + c
    pltpu.sync_copy(zeros_hbm, dacc.at[pl.ds(s * DRPT, DRPT)])
    pltpu.sync_copy(dst_hbm.at[wid], dst_v)
    pltpu.sync_copy(ones_hbm, ones_v)
    plsc.subcore_barrier()

    def body(j, carry):
        @pl.when(j >= 4)
        def _drain():
            pltpu.make_async_copy(ones_v, dacc.at[dst_v.at[j]],
                                  ssem.at[lax.rem(j, 4)]).wait()

        pltpu.async_copy(ones_v, dacc.at[dst_v.at[j]],
                         ssem.at[lax.rem(j, 4)], add=True)
        return carry

    lax.fori_loop(0, NCHUNKD, body, 0)
    for b in range(4):
        pltpu.make_async_copy(ones_v, dacc.at[dst_v.at[0]],
                              ssem.at[b]).wait()
    plsc.subcore_barrier()
    pltpu.sync_copy(dacc.at[pl.ds(s * DRPT, DRPT)],
                    out_hbm.at[c, pl.ds(s * DRPT, DRPT)])


@functools.partial(
    pl.kernel,
    mesh=_mesh,
    out_type=jax.ShapeDtypeStruct((NC, NPAD, D), jnp.float32),
    scratch_types=[
        pltpu.VMEM_SHARED((NPAD, D), jnp.float32),
        pltpu.VMEM((NCHUNK, K), jnp.int32),
        pltpu.VMEM((NCHUNK, K), jnp.int32),
        pltpu.VMEM((5, K, D), jnp.float32),
        pltpu.SemaphoreType.DMA((5,)),
        pltpu.SemaphoreType.DMA((5,)),
    ],
    compiler_params=pltpu.CompilerParams(use_tc_tiling_on_sc=False,
                                         skip_device_barrier=True),
)
def _sc_aggregate(g_hbm, src_hbm, dsti_hbm, zeros_hbm, out_hbm,
                  acc, src_v, dst_v, rows_v, gsem, ssem):
    c = lax.axis_index("c")
    s = lax.axis_index("s")
    wid = s * NC + c
    pltpu.sync_copy(src_hbm.at[wid], src_v)
    pltpu.sync_copy(dsti_hbm.at[wid], dst_v)
    for b in range(4):
        pltpu.async_copy(g_hbm.at[src_v.at[b]], rows_v.at[b], gsem.at[b])

    @pl.when(c == 0)
    def _init_g():
        @pl.when(s < NS - 1)
        def _full():
            pltpu.sync_copy(g_hbm.at[pl.ds(s * RPT, RPT)],
                            acc.at[pl.ds(s * RPT, RPT)])

        @pl.when(s == NS - 1)
        def _tail():
            pltpu.sync_copy(g_hbm.at[pl.ds((NS - 1) * RPT, N - (NS - 1) * RPT)],
                            acc.at[pl.ds((NS - 1) * RPT, N - (NS - 1) * RPT)])
            pltpu.sync_copy(zeros_hbm.at[pl.ds(0, NPAD - N)],
                            acc.at[pl.ds(N, NPAD - N)])

    @pl.when(c == 1)
    def _init_z():
        pltpu.sync_copy(zeros_hbm, acc.at[pl.ds(s * RPT, RPT)])

    plsc.subcore_barrier()

    def body(j, carry):
        p = lax.rem(j, 5)
        nxt = j + 4
        pn = lax.rem(nxt, 5)

        @pl.when(nxt < NCHUNK)
        def _prefetch():
            @pl.when(j >= 1)
            def _drain():
                pltpu.make_async_copy(rows_v.at[pn], acc.at[dst_v.at[j]],
                                      ssem.at[pn]).wait()
            pltpu.async_copy(g_hbm.at[src_v.at[nxt]], rows_v.at[pn],
                             gsem.at[pn])

        pltpu.make_async_copy(g_hbm.at[src_v.at[j]], rows_v.at[p],
                              gsem.at[p]).wait()
        pltpu.async_copy(rows_v.at[p], acc.at[dst_v.at[j]], ssem.at[p],
                         add=True)
        return carry

    lax.fori_loop(0, NCHUNK, body, 0)
    for b in range(5):
        pltpu.make_async_copy(rows_v.at[b], acc.at[dst_v.at[0]],
                              ssem.at[b]).wait()
    plsc.subcore_barrier()
    pltpu.sync_copy(acc.at[pl.ds(s * RPT, RPT)],
                    out_hbm.at[c, pl.ds(s * RPT, RPT)])


_R = 1000


def _tc1_body(x_ref, w_ref, d0_ref, d1_ref, g_ref, dinv_ref):
    deg = d0_ref[0, :, 0:1] + d1_ref[0, :, 0:1] + 1.0
    dinv = lax.rsqrt(deg)
    g_ref[...] = jnp.dot(x_ref[...] * dinv, w_ref[...],
                         preferred_element_type=jnp.float32)
    dinv_ref[...] = dinv


def _tc2_body(a0_ref, a1_ref, dinv_ref, b_ref, w_ref, g2_ref):
    dinv = dinv_ref[...]
    tot = a0_ref[0] + a1_ref[0]
    x2 = jnp.maximum(tot * dinv + b_ref[...], 0.0)
    h2 = jnp.dot(x2, w_ref[...], preferred_element_type=jnp.float32)
    g2_ref[...] = h2 * dinv


def _tc3_body(a0_ref, a1_ref, dinv_ref, b_ref, out_ref):
    tot = a0_ref[0] + a1_ref[0]
    out_ref[...] = tot * dinv_ref[...] + b_ref[...]


def _row_spec(w):
    return pl.BlockSpec((_R, w), lambda i: (i, 0))


def _core_spec(w, c):
    return pl.BlockSpec((1, _R, w), lambda i, _c=c: (_c, i, 0))


_full_spec = pl.BlockSpec((D, D), lambda i: (0, 0))
_bias_spec = pl.BlockSpec((1, D), lambda i: (0, 0))


def kernel(x, edge_index, cache_name, W1, b1, W2, b2):
    src = edge_index[0].reshape(NW, NCHUNK, K)
    dst = edge_index[1].reshape(NW, NCHUNK, K)
    dstd = edge_index[1].reshape(NW, NCHUNKD, KD)
    ones_rows = jnp.ones((KD, DW), jnp.float32)
    zeros_deg = jnp.zeros((DRPT, DW), jnp.float32)
    zeros_agg = jnp.zeros((RPT, D), jnp.float32)
    b1r = b1.reshape(1, D)
    b2r = b2.reshape(1, D)

    dpart = _sc_degree(dstd, ones_rows, zeros_deg)

    g1, dinv = pl.pallas_call(
        _tc1_body,
        grid=(N // _R,),
        in_specs=[_row_spec(D), _full_spec, _core_spec(DW, 0),
                  _core_spec(DW, 1)],
        out_specs=[_row_spec(D), _row_spec(1)],
        out_shape=[jax.ShapeDtypeStruct((N, D), jnp.float32),
                   jax.ShapeDtypeStruct((N, 1), jnp.float32)],
    )(x, W1, dpart, dpart)

    a1 = _sc_aggregate(g1, src, dst, zeros_agg)

    g2 = pl.pallas_call(
        _tc2_body,
        grid=(N // _R,),
        in_specs=[_core_spec(D, 0), _core_spec(D, 1),
                  _row_spec(1), _bias_spec, _full_spec],
        out_specs=_row_spec(D),
        out_shape=jax.ShapeDtypeStruct((N, D), jnp.float32),
    )(a1, a1, dinv, b1r, W2)

    a2 = _sc_aggregate(g2, src, dst, zeros_agg)

    out = pl.pallas_call(
        _tc3_body,
        grid=(N // _R,),
        in_specs=[_core_spec(D, 0), _core_spec(D, 1),
                  _row_spec(1), _bias_spec],
        out_specs=_row_spec(D),
        out_shape=jax.ShapeDtypeStruct((N, D), jnp.float32),
    )(a2, a2, dinv, b2r)

    return out


# speedup vs baseline: 1.0124x; 1.0124x over previous
import functools

import jax
import jax.numpy as jnp
from jax import lax
from jax.experimental import pallas as pl
from jax.experimental.pallas import tpu as pltpu
from jax.experimental.pallas import tpu_sc as plsc

N = 10000
D = 128
E = 320000

NC = 2
NS = 16
NW = NC * NS
EPW = E // NW
K = 40
NCHUNK = EPW // K
NPAD = 10240
RPT = NPAD // NS
DRPT = NPAD // NS
DW = 1
KD = 80
NCHUNKD = EPW // KD

_mesh = plsc.VectorSubcoreMesh(core_axis_name="c", subcore_axis_name="s")


@functools.partial(
    pl.kernel,
    mesh=_mesh,
    out_type=jax.ShapeDtypeStruct((NC, NPAD, DW), jnp.float32),
    scratch_types=[
        pltpu.VMEM_SHARED((NPAD, DW), jnp.float32),
        pltpu.VMEM((NCHUNKD, KD), jnp.int32),
        pltpu.VMEM((KD, DW), jnp.float32),
        pltpu.SemaphoreType.DMA((4,)),
    ],
    compiler_params=pltpu.CompilerParams(use_tc_tiling_on_sc=False),
)
def _sc_degree(dst_hbm, ones_hbm, zeros_hbm, out_hbm, dacc, dst_v, ones_v,
               ssem):
    c = lax.axis_index("c")
    s = lax.axis_index("s")
    wid = s * NC + c
    pltpu.sync_copy(zeros_hbm, dacc.at[pl.ds(s * DRPT, DRPT)])
    pltpu.sync_copy(dst_hbm.at[wid], dst_v)
    pltpu.sync_copy(ones_hbm, ones_v)
    plsc.subcore_barrier()

    def body(j, carry):
        @pl.when(j >= 4)
        def _drain():
            pltpu.make_async_copy(ones_v, dacc.at[dst_v.at[j]],
                                  ssem.at[lax.rem(j, 4)]).wait()

        pltpu.async_copy(ones_v, dacc.at[dst_v.at[j]],
                         ssem.at[lax.rem(j, 4)], add=True)
        return carry

    lax.fori_loop(0, NCHUNKD, body, 0)
    for b in range(4):
        pltpu.make_async_copy(ones_v, dacc.at[dst_v.at[0]],
                              ssem.at[b]).wait()
    plsc.subcore_barrier()
    pltpu.sync_copy(dacc.at[pl.ds(s * DRPT, DRPT)],
                    out_hbm.at[c, pl.ds(s * DRPT, DRPT)])


@functools.partial(
    pl.kernel,
    mesh=_mesh,
    out_type=jax.ShapeDtypeStruct((NC, NPAD, D), jnp.float32),
    scratch_types=[
        pltpu.VMEM_SHARED((NPAD, D), jnp.float32),
        pltpu.VMEM((NCHUNK, K), jnp.int32),
        pltpu.VMEM((NCHUNK, K), jnp.int32),
        pltpu.VMEM((5, K, D), jnp.float32),
        pltpu.SemaphoreType.DMA((5,)),
        pltpu.SemaphoreType.DMA((5,)),
    ],
    compiler_params=pltpu.CompilerParams(use_tc_tiling_on_sc=False),
)
def _sc_aggregate(g_hbm, src_hbm, dsti_hbm, zeros_hbm, out_hbm,
                  acc, src_v, dst_v, rows_v, gsem, ssem):
    c = lax.axis_index("c")
    s = lax.axis_index("s")
    wid = s * NC + c
    pltpu.sync_copy(src_hbm.at[wid], src_v)
    pltpu.sync_copy(dsti_hbm.at[wid], dst_v)
    for b in range(4):
        pltpu.async_copy(g_hbm.at[src_v.at[b]], rows_v.at[b], gsem.at[b])

    @pl.when(c == 0)
    def _init_g():
        @pl.when(s < NS - 1)
        def _full():
            pltpu.sync_copy(g_hbm.at[pl.ds(s * RPT, RPT)],
                            acc.at[pl.ds(s * RPT, RPT)])

        @pl.when(s == NS - 1)
        def _tail():
            pltpu.sync_copy(g_hbm.at[pl.ds((NS - 1) * RPT, N - (NS - 1) * RPT)],
                            acc.at[pl.ds((NS - 1) * RPT, N - (NS - 1) * RPT)])
            pltpu.sync_copy(zeros_hbm.at[pl.ds(0, NPAD - N)],
                            acc.at[pl.ds(N, NPAD - N)])

    @pl.when(c == 1)
    def _init_z():
        pltpu.sync_copy(zeros_hbm, acc.at[pl.ds(s * RPT, RPT)])

    plsc.subcore_barrier()

    def body(j, carry):
        p = lax.rem(j, 5)
        nxt = j + 4
        pn = lax.rem(nxt, 5)

        @pl.when(nxt < NCHUNK)
        def _prefetch():
            @pl.when(j >= 1)
            def _drain():
                pltpu.make_async_copy(rows_v.at[pn], acc.at[dst_v.at[j]],
                                      ssem.at[pn]).wait()
            pltpu.async_copy(g_hbm.at[src_v.at[nxt]], rows_v.at[pn],
                             gsem.at[pn])

        pltpu.make_async_copy(g_hbm.at[src_v.at[j]], rows_v.at[p],
                              gsem.at[p]).wait()
        pltpu.async_copy(rows_v.at[p], acc.at[dst_v.at[j]], ssem.at[p],
                         add=True)
        return carry

    lax.fori_loop(0, NCHUNK, body, 0)
    for b in range(5):
        pltpu.make_async_copy(rows_v.at[b], acc.at[dst_v.at[0]],
                              ssem.at[b]).wait()
    plsc.subcore_barrier()
    pltpu.sync_copy(acc.at[pl.ds(s * RPT, RPT)],
                    out_hbm.at[c, pl.ds(s * RPT, RPT)])


_R = 1000


def _tc1_body(x_ref, w_ref, d0_ref, d1_ref, g_ref, dinv_ref):
    deg = d0_ref[0, :, 0:1] + d1_ref[0, :, 0:1] + 1.0
    dinv = lax.rsqrt(deg)
    g_ref[...] = jnp.dot(x_ref[...] * dinv, w_ref[...],
                         preferred_element_type=jnp.float32)
    dinv_ref[...] = dinv


def _tc2_body(a0_ref, a1_ref, dinv_ref, b_ref, w_ref, g2_ref):
    dinv = dinv_ref[...]
    tot = a0_ref[0] + a1_ref[0]
    x2 = jnp.maximum(tot * dinv + b_ref[...], 0.0)
    h2 = jnp.dot(x2, w_ref[...], preferred_element_type=jnp.float32)
    g2_ref[...] = h2 * dinv


def _tc3_body(a0_ref, a1_ref, dinv_ref, b_ref, out_ref):
    tot = a0_ref[0] + a1_ref[0]
    out_ref[...] = tot * dinv_ref[...] + b_ref[...]


def _row_spec(w):
    return pl.BlockSpec((_R, w), lambda i: (i, 0))


def _core_spec(w, c):
    return pl.BlockSpec((1, _R, w), lambda i, _c=c: (_c, i, 0))


_full_spec = pl.BlockSpec((D, D), lambda i: (0, 0))
_bias_spec = pl.BlockSpec((1, D), lambda i: (0, 0))


def kernel(x, edge_index, cache_name, W1, b1, W2, b2):
    src = edge_index[0].reshape(NW, NCHUNK, K)
    dst = edge_index[1].reshape(NW, NCHUNK, K)
    dstd = edge_index[1].reshape(NW, NCHUNKD, KD)
    ones_rows = jnp.ones((KD, DW), jnp.float32)
    zeros_deg = jnp.zeros((DRPT, DW), jnp.float32)
    zeros_agg = jnp.zeros((RPT, D), jnp.float32)
    b1r = b1.reshape(1, D)
    b2r = b2.reshape(1, D)

    dpart = _sc_degree(dstd, ones_rows, zeros_deg)

    g1, dinv = pl.pallas_call(
        _tc1_body,
        grid=(N // _R,),
        in_specs=[_row_spec(D), _full_spec, _core_spec(DW, 0),
                  _core_spec(DW, 1)],
        out_specs=[_row_spec(D), _row_spec(1)],
        out_shape=[jax.ShapeDtypeStruct((N, D), jnp.float32),
                   jax.ShapeDtypeStruct((N, 1), jnp.float32)],
    )(x, W1, dpart, dpart)

    a1 = _sc_aggregate(g1, src, dst, zeros_agg)

    g2 = pl.pallas_call(
        _tc2_body,
        grid=(N // _R,),
        in_specs=[_core_spec(D, 0), _core_spec(D, 1),
                  _row_spec(1), _bias_spec, _full_spec],
        out_specs=_row_spec(D),
        out_shape=jax.ShapeDtypeStruct((N, D), jnp.float32),
    )(a1, a1, dinv, b1r, W2)

    a2 = _sc_aggregate(g2, src, dst, zeros_agg)

    out = pl.pallas_call(
        _tc3_body,
        grid=(N // _R,),
        in_specs=[_core_spec(D, 0), _core_spec(D, 1),
                  _row_spec(1), _bias_spec],
        out_specs=_row_spec(D),
        out_shape=jax.ShapeDtypeStruct((N, D), jnp.float32),
    )(a2, a2, dinv, b2r)

    return out
